# Initial kernel scaffold; baseline (speedup 1.0000x reference)
import jax
import jax.numpy as jnp
from jax.experimental import pallas as pl


def kernel(walks, lut, sign_l, sign_r):
    raise NotImplementedError("write your pallas kernel here")


# speedup vs baseline: 46.4953x; 46.4953x over previous
import functools

import jax
import jax.numpy as jnp
from jax import lax
from jax.experimental import pallas as pl
from jax.experimental.pallas import tpu as pltpu
from jax.experimental.pallas import tpu_sc as plsc

M = 4096
N = 4096
V = 4
TXS = 16
TYS = 16
LUT_SIZE = 512
W_SCALE = 0.02

NC, NS, L = 2, 16, 16
NW = NC * NS
NBT = M // TXS
TILES_PER_W = NBT // NW
WPT = N * TXS // V
QUADS = N // (V * L)
QUAD_UNROLL = 4


def _sc_body(walks_hbm, lut_hbm, sr_hbm, sl_hbm, out_hbm,
             lutbuf, srbuf, srowv, walksbuf, slbuf, rowbuf):
    wid = lax.axis_index("s") * NC + lax.axis_index("c")
    lane = lax.iota(jnp.int32, L)
    pat_w = (lane // 4) * 64 + (lane % 4)
    pat_s = lane * 4

    pltpu.sync_copy(lut_hbm, lutbuf)
    pltpu.sync_copy(sr_hbm, srbuf)

    def _build_srow(i, c):
        kidx = i * L + lane
        for v in range(V):
            g = plsc.load_gather(srbuf, [kidx * 4 + v])
            srowv[v, pl.ds(i * L, L)] = g * W_SCALE
        return c
    lax.fori_loop(0, N // (V * L), _build_srow, 0)

    def _do_row(tx, bt):
        slsplat = plsc.load_gather(slbuf, [jnp.full((L,), tx, jnp.int32)])

        def _do_quads(i, c):
            for u in range(QUAD_UNROLL):
                jq = i * QUAD_UNROLL + u
                widx = plsc.load_gather(walksbuf, [pat_w + (jq * 256 + tx * 4)])
                tbase = widx * 4
                for v in range(V):
                    g = plsc.load_gather(lutbuf, [tbase + v])
                    s = srowv[v, pl.ds(jq * L, L)]
                    plsc.store_scatter(rowbuf, [pat_s + (jq * 64 + v)],
                                       g * s * slsplat)
            return c
        lax.fori_loop(0, QUADS // QUAD_UNROLL, _do_quads, 0)
        pltpu.sync_copy(rowbuf, out_hbm.at[bt * TXS + tx])
        return bt

    for t in range(TILES_PER_W):
        bt = wid * TILES_PER_W + t
        pltpu.sync_copy(walks_hbm.at[pl.ds(bt * WPT, WPT)], walksbuf)
        pltpu.sync_copy(sl_hbm.at[pl.ds(bt * TXS, TXS)], slbuf)
        lax.fori_loop(0, TXS, _do_row, bt)


@jax.jit
def _sc_dequant(walks, lut_flat, sign_r, sign_l):
    mesh = plsc.VectorSubcoreMesh(core_axis_name="c", subcore_axis_name="s",
                                  num_cores=NC, num_subcores=NS)
    f = pl.kernel(
        _sc_body,
        out_type=jax.ShapeDtypeStruct((M, N), jnp.float32),
        mesh=mesh,
        compiler_params=pltpu.CompilerParams(needs_layout_passes=False),
        scratch_types=[
            pltpu.VMEM((LUT_SIZE * V,), jnp.float32),
            pltpu.VMEM((N,), jnp.float32),
            pltpu.VMEM((V, N // V), jnp.float32),
            pltpu.VMEM((WPT,), jnp.int32),
            pltpu.VMEM((TXS,), jnp.float32),
            pltpu.VMEM((N,), jnp.float32),
        ],
    )
    return f(walks, lut_flat, sign_r, sign_l)


def kernel(walks, lut, sign_l, sign_r):
    walks = walks.astype(jnp.int32)
    lut_flat = lut.reshape(LUT_SIZE * V)
    return _sc_dequant(walks, lut_flat, sign_r, sign_l)


# speedup vs baseline: 152.1281x; 3.2719x over previous
import functools

import jax
import jax.numpy as jnp
from jax import lax
from jax.experimental import pallas as pl
from jax.experimental.pallas import tpu as pltpu
from jax.experimental.pallas import tpu_sc as plsc

M = 4096
N = 4096
V = 4
TXS = 16
TYS = 16
LUT_SIZE = 512
W_SCALE = 0.02

NC, NS, L = 2, 16, 16
NW = NC * NS
NBT = M // TXS
TILES_PER_W = NBT // NW
ROWS_PER_W = TILES_PER_W * TXS
WPT = N * TXS // V
QUADS = N // (V * L)
QUAD_UNROLL = 8


def _sc_body(walks_hbm, lut_hbm, sr_hbm, sl_hbm, out_hbm,
             lutbuf, srbuf, srowv, wb0, wb1, slbuf, rb0, rb1, wsem, rsem):
    wid = lax.axis_index("s") * NC + lax.axis_index("c")
    lane = lax.iota(jnp.int32, L)
    pat_w = (lane // 4) * 64 + (lane % 4)
    pat_s = lane * 4

    pltpu.sync_copy(lut_hbm, lutbuf)
    pltpu.sync_copy(sr_hbm, srbuf)
    pltpu.sync_copy(sl_hbm.at[pl.ds(wid * ROWS_PER_W, ROWS_PER_W)], slbuf)

    def _build_srow(i, c):
        kidx = i * L + lane
        for v in range(V):
            g = plsc.load_gather(srbuf, [kidx * 4 + v])
            srowv[v, pl.ds(i * L, L)] = g * W_SCALE
        return c
    lax.fori_loop(0, N // (V * L), _build_srow, 0)

    for t in range(TILES_PER_W):
        bt = wid * TILES_PER_W + t
        wb = wb0 if t % 2 == 0 else wb1
        if t == 0:
            pltpu.sync_copy(walks_hbm.at[pl.ds(bt * WPT, WPT)], wb)
        if t + 1 < TILES_PER_W:
            pltpu.async_copy(walks_hbm.at[pl.ds((bt + 1) * WPT, WPT)],
                             wb1 if t % 2 == 0 else wb0, wsem)

        def _do_pair(i, c, t=t, wb=wb):
            for sub in range(2):
                tx = i * 2 + sub
                rg = t * TXS + tx
                rb = rb0 if sub == 0 else rb1

                @pl.when(rg >= 2)
                def _wait_prev():
                    pltpu.make_async_copy(rb0, out_hbm.at[0], rsem).wait()

                slsplat = plsc.load_gather(slbuf,
                                           [jnp.full((L,), rg, jnp.int32)])

                @plsc.parallel_loop(0, QUADS, 1, unroll=QUAD_UNROLL)
                def _quad(jq):
                    widx = plsc.load_gather(wb, [pat_w + (jq * 256 + tx * 4)])
                    tbase = widx * 4
                    for v in range(V):
                        g = plsc.load_gather(lutbuf, [tbase + v])
                        s = srowv[v, pl.ds(jq * L, L)]
                        plsc.store_scatter(rb, [pat_s + (jq * 64 + v)],
                                           g * s * slsplat)

                pltpu.async_copy(rb, out_hbm.at[wid * ROWS_PER_W + rg], rsem)
            return c
        lax.fori_loop(0, TXS // 2, _do_pair, 0)

        if t + 1 < TILES_PER_W:
            pltpu.make_async_copy(walks_hbm.at[pl.ds(0, WPT)], wb0, wsem).wait()

    pltpu.make_async_copy(rb0, out_hbm.at[0], rsem).wait()
    pltpu.make_async_copy(rb0, out_hbm.at[0], rsem).wait()


@jax.jit
def _sc_dequant(walks, lut_flat, sign_r, sign_l):
    mesh = plsc.VectorSubcoreMesh(core_axis_name="c", subcore_axis_name="s",
                                  num_cores=NC, num_subcores=NS)
    f = pl.kernel(
        _sc_body,
        out_type=jax.ShapeDtypeStruct((M, N), jnp.float32),
        mesh=mesh,
        compiler_params=pltpu.CompilerParams(needs_layout_passes=False),
        scratch_types=[
            pltpu.VMEM((LUT_SIZE * V,), jnp.float32),
            pltpu.VMEM((N,), jnp.float32),
            pltpu.VMEM((V, N // V), jnp.float32),
            pltpu.VMEM((WPT,), jnp.int32),
            pltpu.VMEM((WPT,), jnp.int32),
            pltpu.VMEM((ROWS_PER_W,), jnp.float32),
            pltpu.VMEM((N,), jnp.float32),
            pltpu.VMEM((N,), jnp.float32),
            pltpu.SemaphoreType.DMA,
            pltpu.SemaphoreType.DMA,
        ],
    )
    return f(walks, lut_flat, sign_r, sign_l)


def kernel(walks, lut, sign_l, sign_r):
    walks = walks.astype(jnp.int32)
    lut_flat = lut.reshape(LUT_SIZE * V)
    return _sc_dequant(walks, lut_flat, sign_r, sign_l)
